# Initial kernel scaffold; baseline (speedup 1.0000x reference)
import jax
import jax.numpy as jnp
import numpy as np
from jax.experimental import pallas as pl
from jax.experimental.pallas import tpu as pltpu

_B, _N, _K, _DP, _DM = 4, 2048, 16, 256, 256
_BLK = 256
_NBLK = _N // _BLK
_GW = 2 * _DM + 3
_HI = jax.lax.Precision.HIGHEST
_DF = jax.lax.Precision.DEFAULT
_INF = np.float32(np.inf)


def _mm(a, b, prec):
    return jax.lax.dot_general(a, b, (((a.ndim - 1,), (0,)), ((), ())),
                               precision=prec,
                               preferred_element_type=jnp.float32)


def _body(xyz_blk_ref, xyz_full_ref, xyzt_ref, feat_full_ref, feat_blk_ref,
          w_fc1_ref, b_fc1_ref, w_fc2_ref, b_fc2_ref,
          w_d1_ref, b_d1_ref, w_d2_ref, b_d2_ref,
          w_g1_ref, b_g1_ref, w_g2_ref, b_g2_ref,
          w_q_ref, w_k_ref, w_v_ref,
          res_ref, attn_ref,
          G_s, qg1_s, wd2g1_s, bg1_s, dist_s, L_s, VPE_s):
    blk = pl.program_id(1)

    @pl.when(blk == 0)
    def _prep():
        w_g1 = w_g1_ref[...]
        x = _mm(feat_full_ref[0], w_fc1_ref[...], _HI) + b_fc1_ref[...]
        qg1_s[...] = _mm(x, _mm(w_q_ref[...], w_g1, _HI), _HI)
        G_s[:, 0:_DM] = _mm(x, _mm(w_k_ref[...], w_g1, _HI), _HI)
        G_s[:, _DM:2 * _DM] = _mm(x, w_v_ref[...], _HI)
        G_s[:, 2 * _DM:_GW] = xyz_full_ref[0]
        wd2g1_s[...] = _mm(w_d2_ref[...], w_g1, _HI)
        bg1_s[...] = _mm(b_d2_ref[...], w_g1, _HI) + b_g1_ref[...]

    xyz_blk = xyz_blk_ref[0]
    xyzt = xyzt_ref[0]
    sq_all = jnp.sum(xyzt * xyzt, axis=0, keepdims=True)
    sq_blk = jnp.sum(xyz_blk * xyz_blk, axis=1, keepdims=True)
    dist_s[...] = (sq_blk + sq_all) - 2.0 * _mm(xyz_blk, xyzt, _HI)

    col = jax.lax.broadcasted_iota(jnp.int32, (_BLK, _N), 1)
    qg1 = qg1_s[pl.ds(blk * _BLK, _BLK), :]
    bg1 = bg1_s[...]
    w_d1 = w_d1_ref[...]
    b_d1 = b_d1_ref[...]
    w_d2 = w_d2_ref[...]
    b_d2 = b_d2_ref[...]
    wd2g1 = wd2g1_s[...]
    w_g2 = w_g2_ref[...]
    b_g2 = b_g2_ref[...]

    m = None
    for t in range(_K):
        d = dist_s[...]
        dmin = jnp.min(d, axis=1, keepdims=True)
        idx = jnp.min(jnp.where(d == dmin, col, _N), axis=1, keepdims=True)
        oh_b = col == idx
        dist_s[...] = jnp.where(oh_b, _INF, d)
        g = _mm(oh_b.astype(jnp.float32), G_s[...], _DF)
        kg1_t = g[:, 0:_DM]
        vv_t = g[:, _DM:2 * _DM]
        xyz_t = g[:, 2 * _DM:_GW]
        rel = xyz_blk - xyz_t
        h = jnp.maximum(_mm(rel, w_d1, _HI) + b_d1, 0.0)
        pe = _mm(h, w_d2, _HI) + b_d2
        peg1 = _mm(h, wd2g1, _HI)
        inner = jnp.maximum(qg1 - kg1_t + peg1 + bg1, 0.0)
        l_t = _mm(inner, w_g2, _HI) + b_g2
        L_s[t] = l_t
        VPE_s[t] = vv_t + pe
        m = l_t if m is None else jnp.maximum(m, l_t)

    inv = np.float32(1.0 / np.sqrt(_DM))
    s = jnp.zeros((_BLK, _DM), jnp.float32)
    for t in range(_K):
        p = jnp.exp((L_s[t] - m) * inv)
        L_s[t] = p
        s = s + p
    rs = 1.0 / s
    racc = jnp.zeros((_BLK, _DM), jnp.float32)
    for t in range(_K):
        a_t = L_s[t] * rs
        attn_ref[0, :, t, :] = a_t
        racc = racc + a_t * VPE_s[t]
    res_ref[0] = _mm(racc, w_fc2_ref[...], _HI) + b_fc2_ref[...] + feat_blk_ref[0]


def kernel(xyz, features, w_fc1, b_fc1, w_fc2, b_fc2, w_d1, b_d1, w_d2, b_d2,
           w_g1, b_g1, w_g2, b_g2, w_q, w_k, w_v):
    xyzt = jnp.swapaxes(xyz, 1, 2)
    r2 = lambda v: v.reshape(1, -1)

    def wspec(shape):
        return pl.BlockSpec(shape, lambda b, j: (0,) * len(shape))

    in_specs = [
        pl.BlockSpec((1, _BLK, 3), lambda b, j: (b, j, 0)),
        pl.BlockSpec((1, _N, 3), lambda b, j: (b, 0, 0)),
        pl.BlockSpec((1, 3, _N), lambda b, j: (b, 0, 0)),
        pl.BlockSpec((1, _N, _DP), lambda b, j: (b, 0, 0)),
        pl.BlockSpec((1, _BLK, _DP), lambda b, j: (b, j, 0)),
        wspec((_DP, _DM)), wspec((1, _DM)),
        wspec((_DM, _DP)), wspec((1, _DP)),
        wspec((3, _DM)), wspec((1, _DM)),
        wspec((_DM, _DM)), wspec((1, _DM)),
        wspec((_DM, _DM)), wspec((1, _DM)),
        wspec((_DM, _DM)), wspec((1, _DM)),
        wspec((_DM, _DM)),
        wspec((_DM, _DM)),
        wspec((_DM, _DM)),
    ]
    out_specs = [
        pl.BlockSpec((1, _BLK, _DP), lambda b, j: (b, j, 0)),
        pl.BlockSpec((1, _BLK, _K, _DM), lambda b, j: (b, j, 0, 0)),
    ]
    out_shape = [
        jax.ShapeDtypeStruct((_B, _N, _DP), jnp.float32),
        jax.ShapeDtypeStruct((_B, _N, _K, _DM), jnp.float32),
    ]
    scratch_shapes = [
        pltpu.VMEM((_N, _GW), jnp.float32),
        pltpu.VMEM((_N, _DM), jnp.float32),
        pltpu.VMEM((_DM, _DM), jnp.float32),
        pltpu.VMEM((1, _DM), jnp.float32),
        pltpu.VMEM((_BLK, _N), jnp.float32),
        pltpu.VMEM((_K, _BLK, _DM), jnp.float32),
        pltpu.VMEM((_K, _BLK, _DM), jnp.float32),
    ]
    res, attn = pl.pallas_call(
        _body,
        grid=(_B, _NBLK),
        in_specs=in_specs,
        out_specs=out_specs,
        out_shape=out_shape,
        scratch_shapes=scratch_shapes,
        compiler_params=pltpu.CompilerParams(
            dimension_semantics=("arbitrary", "arbitrary")),
    )(xyz, xyz, xyzt, features,
      features, w_fc1, r2(b_fc1), w_fc2, r2(b_fc2), w_d1, r2(b_d1),
      w_d2, r2(b_d2), w_g1, r2(b_g1), w_g2, r2(b_g2), w_q, w_k, w_v)
    return res, attn


# speedup vs baseline: 5.8772x; 5.8772x over previous
import jax
import jax.numpy as jnp
import numpy as np
from jax.experimental import pallas as pl
from jax.experimental.pallas import tpu as pltpu

_B, _N, _K, _DP, _DM = 4, 2048, 16, 256, 256
_BLK = 256
_NBLK = _N // _BLK
_GW = 2 * _DM + 3
_HI = jax.lax.Precision.HIGHEST
_DF = jax.lax.Precision.DEFAULT
_INF = np.float32(np.inf)


def _mm(a, b, prec):
    return jax.lax.dot_general(a, b, (((a.ndim - 1,), (0,)), ((), ())),
                               precision=prec,
                               preferred_element_type=jnp.float32)


def _body(xyz_blk_ref, xyz_full_ref, xyzt_ref, feat_full_ref, feat_blk_ref,
          w_fc1_ref, b_fc1_ref, w_fc2_ref, b_fc2_ref,
          w_d1_ref, b_d1_ref, w_d2_ref, b_d2_ref,
          w_g1_ref, b_g1_ref, w_g2_ref, b_g2_ref,
          w_q_ref, w_k_ref, w_v_ref,
          res_ref, attn_ref,
          G_s, qg1_s, wd2g1_s, bg1_s, dist_s, L_s, VPE_s):
    blk = pl.program_id(1)

    @pl.when(blk == 0)
    def _prep():
        w_g1 = w_g1_ref[...]
        x = _mm(feat_full_ref[0], w_fc1_ref[...], _HI) + b_fc1_ref[...]
        qg1_s[...] = _mm(x, _mm(w_q_ref[...], w_g1, _HI), _HI)
        G_s[:, 0:_DM] = _mm(x, _mm(w_k_ref[...], w_g1, _HI), _HI)
        G_s[:, _DM:2 * _DM] = _mm(x, w_v_ref[...], _HI)
        G_s[:, 2 * _DM:_GW] = xyz_full_ref[0]
        wd2g1_s[...] = _mm(w_d2_ref[...], w_g1, _HI)
        bg1_s[...] = _mm(b_d2_ref[...], w_g1, _HI) + b_g1_ref[...]

    xyz_blk = xyz_blk_ref[0]
    xyzt = xyzt_ref[0]
    sq_all = jnp.sum(xyzt * xyzt, axis=0, keepdims=True)
    sq_blk = jnp.sum(xyz_blk * xyz_blk, axis=1, keepdims=True)
    dist_s[...] = (sq_blk + sq_all) - 2.0 * _mm(xyz_blk, xyzt, _DF)

    col = jax.lax.broadcasted_iota(jnp.int32, (_BLK, _N), 1)
    qg1 = qg1_s[pl.ds(blk * _BLK, _BLK), :]
    bg1 = bg1_s[...]
    w_d1 = w_d1_ref[...]
    b_d1 = b_d1_ref[...]
    w_d2 = w_d2_ref[...]
    b_d2 = b_d2_ref[...]
    wd2g1 = wd2g1_s[...]
    w_g2 = w_g2_ref[...]
    b_g2 = b_g2_ref[...]

    m = None
    for t in range(_K):
        d = dist_s[...]
        dmin = jnp.min(d, axis=1, keepdims=True)
        idx = jnp.min(jnp.where(d == dmin, col, _N), axis=1, keepdims=True)
        oh_b = col == idx
        dist_s[...] = jnp.where(oh_b, _INF, d)
        g = _mm(oh_b.astype(jnp.float32), G_s[...], _DF)
        kg1_t = g[:, 0:_DM]
        vv_t = g[:, _DM:2 * _DM]
        xyz_t = g[:, 2 * _DM:_GW]
        rel = xyz_blk - xyz_t
        h = jnp.maximum(_mm(rel, w_d1, _HI) + b_d1, 0.0)
        pe = _mm(h, w_d2, _HI) + b_d2
        peg1 = _mm(h, wd2g1, _HI)
        inner = jnp.maximum(qg1 - kg1_t + peg1 + bg1, 0.0)
        l_t = _mm(inner, w_g2, _HI) + b_g2
        L_s[t] = l_t
        VPE_s[t] = vv_t + pe
        m = l_t if m is None else jnp.maximum(m, l_t)

    inv = np.float32(1.0 / np.sqrt(_DM))
    s = jnp.zeros((_BLK, _DM), jnp.float32)
    for t in range(_K):
        p = jnp.exp((L_s[t] - m) * inv)
        L_s[t] = p
        s = s + p
    rs = 1.0 / s
    racc = jnp.zeros((_BLK, _DM), jnp.float32)
    for t in range(_K):
        a_t = L_s[t] * rs
        attn_ref[0, :, t, :] = a_t
        racc = racc + a_t * VPE_s[t]
    res_ref[0] = _mm(racc, w_fc2_ref[...], _HI) + b_fc2_ref[...] + feat_blk_ref[0]


def kernel(xyz, features, w_fc1, b_fc1, w_fc2, b_fc2, w_d1, b_d1, w_d2, b_d2,
           w_g1, b_g1, w_g2, b_g2, w_q, w_k, w_v):
    xyzt = jnp.swapaxes(xyz, 1, 2)
    r2 = lambda v: v.reshape(1, -1)

    def wspec(shape):
        return pl.BlockSpec(shape, lambda b, j: (0,) * len(shape))

    in_specs = [
        pl.BlockSpec((1, _BLK, 3), lambda b, j: (b, j, 0)),
        pl.BlockSpec((1, _N, 3), lambda b, j: (b, 0, 0)),
        pl.BlockSpec((1, 3, _N), lambda b, j: (b, 0, 0)),
        pl.BlockSpec((1, _N, _DP), lambda b, j: (b, 0, 0)),
        pl.BlockSpec((1, _BLK, _DP), lambda b, j: (b, j, 0)),
        wspec((_DP, _DM)), wspec((1, _DM)),
        wspec((_DM, _DP)), wspec((1, _DP)),
        wspec((3, _DM)), wspec((1, _DM)),
        wspec((_DM, _DM)), wspec((1, _DM)),
        wspec((_DM, _DM)), wspec((1, _DM)),
        wspec((_DM, _DM)), wspec((1, _DM)),
        wspec((_DM, _DM)),
        wspec((_DM, _DM)),
        wspec((_DM, _DM)),
    ]
    out_specs = [
        pl.BlockSpec((1, _BLK, _DP), lambda b, j: (b, j, 0)),
        pl.BlockSpec((1, _BLK, _K, _DM), lambda b, j: (b, j, 0, 0)),
    ]
    out_shape = [
        jax.ShapeDtypeStruct((_B, _N, _DP), jnp.float32),
        jax.ShapeDtypeStruct((_B, _N, _K, _DM), jnp.float32),
    ]
    scratch_shapes = [
        pltpu.VMEM((_N, _GW), jnp.float32),
        pltpu.VMEM((_N, _DM), jnp.float32),
        pltpu.VMEM((_DM, _DM), jnp.float32),
        pltpu.VMEM((1, _DM), jnp.float32),
        pltpu.VMEM((_BLK, _N), jnp.float32),
        pltpu.VMEM((_K, _BLK, _DM), jnp.float32),
        pltpu.VMEM((_K, _BLK, _DM), jnp.float32),
    ]
    res, attn = pl.pallas_call(
        _body,
        grid=(_B, _NBLK),
        in_specs=in_specs,
        out_specs=out_specs,
        out_shape=out_shape,
        scratch_shapes=scratch_shapes,
        compiler_params=pltpu.CompilerParams(
            dimension_semantics=("arbitrary", "arbitrary")),
    )(xyz, xyz, xyzt, features,
      features, w_fc1, r2(b_fc1), w_fc2, r2(b_fc2), w_d1, r2(b_d1),
      w_d2, r2(b_d2), w_g1, r2(b_g1), w_g2, r2(b_g2), w_q, w_k, w_v)
    return res, attn


# speedup vs baseline: 7.5955x; 1.2924x over previous
import jax
import jax.numpy as jnp
import numpy as np
from jax.experimental import pallas as pl
from jax.experimental.pallas import tpu as pltpu

_B, _N, _K, _DP, _DM = 4, 2048, 16, 256, 256
_BLK = 256
_NBLK = _N // _BLK
_R = _K * _BLK
_GW = 2 * _DM + 6
_HI = jax.lax.Precision.HIGHEST
_DF = jax.lax.Precision.DEFAULT
_INF = np.float32(np.inf)


def _mm(a, b, prec):
    return jax.lax.dot_general(a, b, (((a.ndim - 1,), (0,)), ((), ())),
                               precision=prec,
                               preferred_element_type=jnp.float32)


def _body(xyz_blk_ref, xyz_full_ref, xyzt_ref, feat_full_ref, feat_blk_ref,
          w_fc1_ref, b_fc1_ref, w_fc2_ref, b_fc2_ref,
          w_d1_ref, b_d1_ref, w_d2_ref, b_d2_ref,
          w_g1_ref, b_g1_ref, w_g2_ref, b_g2_ref,
          w_q_ref, w_k_ref, w_v_ref,
          res_ref, attn_ref,
          G_s, qg1_s, wd2g1_s, bg1_s, dist_s, OH_s, L_s, VPE_s):
    blk = pl.program_id(1)

    @pl.when(blk == 0)
    def _prep():
        w_g1 = w_g1_ref[...]
        x = _mm(feat_full_ref[0], w_fc1_ref[...], _HI) + b_fc1_ref[...]
        qg1_s[...] = _mm(x, _mm(w_q_ref[...], w_g1, _HI), _HI)
        G_s[:, 0:_DM] = _mm(x, _mm(w_k_ref[...], w_g1, _HI), _HI).astype(jnp.bfloat16)
        G_s[:, _DM:2 * _DM] = _mm(x, w_v_ref[...], _HI).astype(jnp.bfloat16)
        xyz_f = xyz_full_ref[0]
        xyz_hi = xyz_f.astype(jnp.bfloat16)
        G_s[:, 2 * _DM:2 * _DM + 3] = xyz_hi
        G_s[:, 2 * _DM + 3:_GW] = (xyz_f - xyz_hi.astype(jnp.float32)
                                   ).astype(jnp.bfloat16)
        wd2g1_s[...] = _mm(w_d2_ref[...], w_g1, _HI)
        bg1_s[...] = _mm(b_d2_ref[...], w_g1, _HI) + b_g1_ref[...]

    xyz_blk = xyz_blk_ref[0]
    xyzt = xyzt_ref[0]
    sq_all = jnp.sum(xyzt * xyzt, axis=0, keepdims=True)
    sq_blk = jnp.sum(xyz_blk * xyz_blk, axis=1, keepdims=True)
    dist_s[...] = (sq_blk + sq_all) - 2.0 * _mm(xyz_blk, xyzt, _DF)

    col = jax.lax.broadcasted_iota(jnp.int32, (_BLK, _N), 1)
    for t in range(_K):
        d = dist_s[...]
        dmin = jnp.min(d, axis=1, keepdims=True)
        idx = jnp.min(jnp.where(d == dmin, col, _N), axis=1, keepdims=True)
        oh_b = col == idx
        dist_s[...] = jnp.where(oh_b, _INF, d)
        OH_s[pl.ds(t * _BLK, _BLK), :] = oh_b.astype(jnp.bfloat16)

    g = _mm(OH_s[...], G_s[...], _DF)
    kg1 = g[:, 0:_DM]
    vv = g[:, _DM:2 * _DM]
    xyz_g = g[:, 2 * _DM:2 * _DM + 3] + g[:, 2 * _DM + 3:_GW]
    rel = jnp.broadcast_to(xyz_blk[None], (_K, _BLK, 3)).reshape(_R, 3) - xyz_g
    h = jnp.maximum(_mm(rel, w_d1_ref[...], _DF) + b_d1_ref[...], 0.0)
    pe = _mm(h, w_d2_ref[...], _DF) + b_d2_ref[...]
    peg1 = _mm(h, wd2g1_s[...], _DF)
    qg1 = qg1_s[pl.ds(blk * _BLK, _BLK), :]
    qg1t = jnp.broadcast_to(qg1[None], (_K, _BLK, _DM)).reshape(_R, _DM)
    inner = jnp.maximum(qg1t - kg1 + peg1 + bg1_s[...], 0.0)
    L_s[...] = _mm(inner, w_g2_ref[...], _DF) + b_g2_ref[...]
    VPE_s[...] = vv + pe

    m = None
    for t in range(_K):
        l_t = L_s[pl.ds(t * _BLK, _BLK), :]
        m = l_t if m is None else jnp.maximum(m, l_t)
    inv = np.float32(1.0 / np.sqrt(_DM))
    s = jnp.zeros((_BLK, _DM), jnp.float32)
    for t in range(_K):
        p = jnp.exp((L_s[pl.ds(t * _BLK, _BLK), :] - m) * inv)
        L_s[pl.ds(t * _BLK, _BLK), :] = p
        s = s + p
    rs = 1.0 / s
    racc = jnp.zeros((_BLK, _DM), jnp.float32)
    for t in range(_K):
        a_t = L_s[pl.ds(t * _BLK, _BLK), :] * rs
        attn_ref[0, :, t, :] = a_t
        racc = racc + a_t * VPE_s[pl.ds(t * _BLK, _BLK), :]
    res_ref[0] = _mm(racc, w_fc2_ref[...], _DF) + b_fc2_ref[...] + feat_blk_ref[0]


def kernel(xyz, features, w_fc1, b_fc1, w_fc2, b_fc2, w_d1, b_d1, w_d2, b_d2,
           w_g1, b_g1, w_g2, b_g2, w_q, w_k, w_v):
    xyzt = jnp.swapaxes(xyz, 1, 2)
    r2 = lambda v: v.reshape(1, -1)

    def wspec(shape):
        return pl.BlockSpec(shape, lambda b, j: (0,) * len(shape))

    in_specs = [
        pl.BlockSpec((1, _BLK, 3), lambda b, j: (b, j, 0)),
        pl.BlockSpec((1, _N, 3), lambda b, j: (b, 0, 0)),
        pl.BlockSpec((1, 3, _N), lambda b, j: (b, 0, 0)),
        pl.BlockSpec((1, _N, _DP), lambda b, j: (b, 0, 0)),
        pl.BlockSpec((1, _BLK, _DP), lambda b, j: (b, j, 0)),
        wspec((_DP, _DM)), wspec((1, _DM)),
        wspec((_DM, _DP)), wspec((1, _DP)),
        wspec((3, _DM)), wspec((1, _DM)),
        wspec((_DM, _DM)), wspec((1, _DM)),
        wspec((_DM, _DM)), wspec((1, _DM)),
        wspec((_DM, _DM)), wspec((1, _DM)),
        wspec((_DM, _DM)),
        wspec((_DM, _DM)),
        wspec((_DM, _DM)),
    ]
    out_specs = [
        pl.BlockSpec((1, _BLK, _DP), lambda b, j: (b, j, 0)),
        pl.BlockSpec((1, _BLK, _K, _DM), lambda b, j: (b, j, 0, 0)),
    ]
    out_shape = [
        jax.ShapeDtypeStruct((_B, _N, _DP), jnp.float32),
        jax.ShapeDtypeStruct((_B, _N, _K, _DM), jnp.float32),
    ]
    scratch_shapes = [
        pltpu.VMEM((_N, _GW), jnp.bfloat16),
        pltpu.VMEM((_N, _DM), jnp.float32),
        pltpu.VMEM((_DM, _DM), jnp.float32),
        pltpu.VMEM((1, _DM), jnp.float32),
        pltpu.VMEM((_BLK, _N), jnp.float32),
        pltpu.VMEM((_R, _N), jnp.bfloat16),
        pltpu.VMEM((_R, _DM), jnp.float32),
        pltpu.VMEM((_R, _DM), jnp.float32),
    ]
    res, attn = pl.pallas_call(
        _body,
        grid=(_B, _NBLK),
        in_specs=in_specs,
        out_specs=out_specs,
        out_shape=out_shape,
        scratch_shapes=scratch_shapes,
        compiler_params=pltpu.CompilerParams(
            dimension_semantics=("arbitrary", "arbitrary"),
            vmem_limit_bytes=64 * 1024 * 1024),
    )(xyz, xyz, xyzt, features,
      features, w_fc1, r2(b_fc1), w_fc2, r2(b_fc2), w_d1, r2(b_d1),
      w_d2, r2(b_d2), w_g1, r2(b_g1), w_g2, r2(b_g2), w_q, w_k, w_v)
    return res, attn


# speedup vs baseline: 7.8345x; 1.0315x over previous
import jax
import jax.numpy as jnp
import numpy as np
from jax.experimental import pallas as pl
from jax.experimental.pallas import tpu as pltpu

_B, _N, _K, _DP, _DM = 4, 2048, 16, 256, 256
_BLK = 256
_NBLK = _N // _BLK
_R = _K * _BLK
_GW = 2 * _DM + 6
_HI = jax.lax.Precision.HIGHEST
_DF = jax.lax.Precision.DEFAULT
_INF = np.float32(np.inf)


def _mm(a, b, prec):
    return jax.lax.dot_general(a, b, (((a.ndim - 1,), (0,)), ((), ())),
                               precision=prec,
                               preferred_element_type=jnp.float32)


def _body(xyz_blk_ref, xyz_full_ref, xyzt_ref, feat_full_ref, feat_blk_ref,
          w_fc1_ref, b_fc1_ref, w_fc2_ref, b_fc2_ref,
          w_d1_ref, b_d1_ref, w_d2_ref, b_d2_ref,
          w_g1_ref, b_g1_ref, w_g2_ref, b_g2_ref,
          w_q_ref, w_k_ref, w_v_ref,
          res_ref, attn_ref,
          G_s, qg1_s, wd2g1_s, bg1_s, dist_s, OH_s, L_s, VPE_s):
    blk = pl.program_id(1)

    @pl.when(blk == 0)
    def _prep():
        w_g1 = w_g1_ref[...]
        x = _mm(feat_full_ref[0], w_fc1_ref[...], _HI) + b_fc1_ref[...]
        qg1_s[...] = _mm(x, _mm(w_q_ref[...], w_g1, _HI), _HI)
        G_s[:, 0:_DM] = _mm(x, _mm(w_k_ref[...], w_g1, _HI), _HI).astype(jnp.bfloat16)
        G_s[:, _DM:2 * _DM] = _mm(x, w_v_ref[...], _HI).astype(jnp.bfloat16)
        xyz_f = xyz_full_ref[0]
        xyz_hi = xyz_f.astype(jnp.bfloat16)
        G_s[:, 2 * _DM:2 * _DM + 3] = xyz_hi
        G_s[:, 2 * _DM + 3:_GW] = (xyz_f - xyz_hi.astype(jnp.float32)
                                   ).astype(jnp.bfloat16)
        wd2g1_s[...] = _mm(w_d2_ref[...], w_g1, _HI)
        bg1_s[...] = _mm(b_d2_ref[...], w_g1, _HI) + b_g1_ref[...]

    xyz_blk = xyz_blk_ref[0]
    xyzt = xyzt_ref[0]
    sq_all = jnp.sum(xyzt * xyzt, axis=0, keepdims=True)
    sq_blk = jnp.sum(xyz_blk * xyz_blk, axis=1, keepdims=True)
    dist_s[...] = (sq_blk + sq_all) - 2.0 * _mm(xyz_blk, xyzt, _DF)

    col = jax.lax.broadcasted_iota(jnp.int32, (_BLK, _N), 1)
    for t in range(_K):
        d = dist_s[...]
        idx = jnp.argmin(d, axis=1)[:, None]
        oh_b = col == idx
        dist_s[...] = jnp.where(oh_b, _INF, d)
        OH_s[pl.ds(t * _BLK, _BLK), :] = oh_b.astype(jnp.bfloat16)

    g = _mm(OH_s[...], G_s[...], _DF)
    kg1 = g[:, 0:_DM]
    vv = g[:, _DM:2 * _DM]
    xyz_g = g[:, 2 * _DM:2 * _DM + 3] + g[:, 2 * _DM + 3:_GW]
    rel = jnp.broadcast_to(xyz_blk[None], (_K, _BLK, 3)).reshape(_R, 3) - xyz_g
    h = jnp.maximum(_mm(rel, w_d1_ref[...], _DF) + b_d1_ref[...], 0.0)
    pe = _mm(h, w_d2_ref[...], _DF) + b_d2_ref[...]
    peg1 = _mm(h, wd2g1_s[...], _DF)
    qg1 = qg1_s[pl.ds(blk * _BLK, _BLK), :]
    qg1t = jnp.broadcast_to(qg1[None], (_K, _BLK, _DM)).reshape(_R, _DM)
    inner = jnp.maximum(qg1t - kg1 + peg1 + bg1_s[...], 0.0)
    L_s[...] = _mm(inner, w_g2_ref[...], _DF) + b_g2_ref[...]
    VPE_s[...] = vv + pe

    m = None
    for t in range(_K):
        l_t = L_s[pl.ds(t * _BLK, _BLK), :]
        m = l_t if m is None else jnp.maximum(m, l_t)
    inv = np.float32(1.0 / np.sqrt(_DM))
    s = jnp.zeros((_BLK, _DM), jnp.float32)
    for t in range(_K):
        p = jnp.exp((L_s[pl.ds(t * _BLK, _BLK), :] - m) * inv)
        L_s[pl.ds(t * _BLK, _BLK), :] = p
        s = s + p
    rs = 1.0 / s
    racc = jnp.zeros((_BLK, _DM), jnp.float32)
    for t in range(_K):
        a_t = L_s[pl.ds(t * _BLK, _BLK), :] * rs
        attn_ref[0, :, t, :] = a_t
        racc = racc + a_t * VPE_s[pl.ds(t * _BLK, _BLK), :]
    res_ref[0] = _mm(racc, w_fc2_ref[...], _DF) + b_fc2_ref[...] + feat_blk_ref[0]


def kernel(xyz, features, w_fc1, b_fc1, w_fc2, b_fc2, w_d1, b_d1, w_d2, b_d2,
           w_g1, b_g1, w_g2, b_g2, w_q, w_k, w_v):
    xyzt = jnp.swapaxes(xyz, 1, 2)
    r2 = lambda v: v.reshape(1, -1)

    def wspec(shape):
        return pl.BlockSpec(shape, lambda b, j: (0,) * len(shape))

    in_specs = [
        pl.BlockSpec((1, _BLK, 3), lambda b, j: (b, j, 0)),
        pl.BlockSpec((1, _N, 3), lambda b, j: (b, 0, 0)),
        pl.BlockSpec((1, 3, _N), lambda b, j: (b, 0, 0)),
        pl.BlockSpec((1, _N, _DP), lambda b, j: (b, 0, 0)),
        pl.BlockSpec((1, _BLK, _DP), lambda b, j: (b, j, 0)),
        wspec((_DP, _DM)), wspec((1, _DM)),
        wspec((_DM, _DP)), wspec((1, _DP)),
        wspec((3, _DM)), wspec((1, _DM)),
        wspec((_DM, _DM)), wspec((1, _DM)),
        wspec((_DM, _DM)), wspec((1, _DM)),
        wspec((_DM, _DM)), wspec((1, _DM)),
        wspec((_DM, _DM)),
        wspec((_DM, _DM)),
        wspec((_DM, _DM)),
    ]
    out_specs = [
        pl.BlockSpec((1, _BLK, _DP), lambda b, j: (b, j, 0)),
        pl.BlockSpec((1, _BLK, _K, _DM), lambda b, j: (b, j, 0, 0)),
    ]
    out_shape = [
        jax.ShapeDtypeStruct((_B, _N, _DP), jnp.float32),
        jax.ShapeDtypeStruct((_B, _N, _K, _DM), jnp.float32),
    ]
    scratch_shapes = [
        pltpu.VMEM((_N, _GW), jnp.bfloat16),
        pltpu.VMEM((_N, _DM), jnp.float32),
        pltpu.VMEM((_DM, _DM), jnp.float32),
        pltpu.VMEM((1, _DM), jnp.float32),
        pltpu.VMEM((_BLK, _N), jnp.float32),
        pltpu.VMEM((_R, _N), jnp.bfloat16),
        pltpu.VMEM((_R, _DM), jnp.float32),
        pltpu.VMEM((_R, _DM), jnp.float32),
    ]
    res, attn = pl.pallas_call(
        _body,
        grid=(_B, _NBLK),
        in_specs=in_specs,
        out_specs=out_specs,
        out_shape=out_shape,
        scratch_shapes=scratch_shapes,
        compiler_params=pltpu.CompilerParams(
            dimension_semantics=("arbitrary", "arbitrary"),
            vmem_limit_bytes=64 * 1024 * 1024),
    )(xyz, xyz, xyzt, features,
      features, w_fc1, r2(b_fc1), w_fc2, r2(b_fc2), w_d1, r2(b_d1),
      w_d2, r2(b_d2), w_g1, r2(b_g1), w_g2, r2(b_g2), w_q, w_k, w_v)
    return res, attn


# speedup vs baseline: 7.8397x; 1.0007x over previous
import jax
import jax.numpy as jnp
import numpy as np
from jax.experimental import pallas as pl
from jax.experimental.pallas import tpu as pltpu

_B, _N, _K, _DP, _DM = 4, 2048, 16, 256, 256
_BLK = 256
_NBLK = _N // _BLK
_R = _K * _BLK
_GW = 2 * _DM + 6
_HI = jax.lax.Precision.HIGHEST
_DF = jax.lax.Precision.DEFAULT
_INF = np.float32(np.inf)


def _mm(a, b, prec):
    return jax.lax.dot_general(a, b, (((a.ndim - 1,), (0,)), ((), ())),
                               precision=prec,
                               preferred_element_type=jnp.float32)


def _body(xyz_blk_ref, xyz_full_ref, xyzt_ref, feat_full_ref, feat_blk_ref,
          w_fc1_ref, b_fc1_ref, w_fc2_ref, b_fc2_ref,
          w_d1_ref, b_d1_ref, w_d2_ref, b_d2_ref,
          w_g1_ref, b_g1_ref, w_g2_ref, b_g2_ref,
          w_q_ref, w_k_ref, w_v_ref,
          res_ref, attn_ref,
          G_s, qg1_s, wd2g1_s, bg1_s, dist_s, OH_s, L_s, VPE_s):
    blk = pl.program_id(1)

    @pl.when(blk == 0)
    def _prep():
        w_g1 = w_g1_ref[...]
        x = _mm(feat_full_ref[0], w_fc1_ref[...], _HI) + b_fc1_ref[...]
        qg1_s[...] = _mm(x, _mm(w_q_ref[...], w_g1, _HI), _HI)
        G_s[:, 0:_DM] = _mm(x, _mm(w_k_ref[...], w_g1, _HI), _HI).astype(jnp.bfloat16)
        G_s[:, _DM:2 * _DM] = _mm(x, w_v_ref[...], _HI).astype(jnp.bfloat16)
        xyz_f = xyz_full_ref[0]
        xyz_hi = xyz_f.astype(jnp.bfloat16)
        G_s[:, 2 * _DM:2 * _DM + 3] = xyz_hi
        G_s[:, 2 * _DM + 3:_GW] = (xyz_f - xyz_hi.astype(jnp.float32)
                                   ).astype(jnp.bfloat16)
        wd2g1_s[...] = _mm(w_d2_ref[...], w_g1, _HI)
        bg1_s[...] = _mm(b_d2_ref[...], w_g1, _HI) + b_g1_ref[...]

    xyz_blk = xyz_blk_ref[0]
    xyzt = xyzt_ref[0]
    sq_all = jnp.sum(xyzt * xyzt, axis=0, keepdims=True)
    sq_blk = jnp.sum(xyz_blk * xyz_blk, axis=1, keepdims=True)
    dist_s[...] = (sq_blk + sq_all) - 2.0 * _mm(xyz_blk, xyzt, _DF)

    col = jax.lax.broadcasted_iota(jnp.int32, (_BLK, _N), 1)
    for t in range(_K):
        d = dist_s[...]
        idx = jnp.argmin(d, axis=1)[:, None]
        oh_b = col == idx
        dist_s[...] = jnp.where(oh_b, _INF, d)
        OH_s[pl.ds(t * _BLK, _BLK), :] = oh_b.astype(jnp.bfloat16)

    g = _mm(OH_s[...], G_s[...], _DF)
    kg1 = g[:, 0:_DM]
    vv = g[:, _DM:2 * _DM]
    xyz_g = g[:, 2 * _DM:2 * _DM + 3] + g[:, 2 * _DM + 3:_GW]
    rel = jnp.broadcast_to(xyz_blk[None], (_K, _BLK, 3)).reshape(_R, 3) - xyz_g
    h = jnp.maximum(_mm(rel, w_d1_ref[...], _DF) + b_d1_ref[...], 0.0)
    pe = _mm(h, w_d2_ref[...], _DF) + b_d2_ref[...]
    peg1 = _mm(h, wd2g1_s[...], _DF)
    qg1 = qg1_s[pl.ds(blk * _BLK, _BLK), :]
    qg1t = jnp.broadcast_to(qg1[None], (_K, _BLK, _DM)).reshape(_R, _DM)
    inner = jnp.maximum(qg1t - kg1 + peg1 + bg1_s[...], 0.0)
    l_all = _mm(inner, w_g2_ref[...], _DF) + b_g2_ref[...]
    L_s[...] = l_all
    m = jnp.max(l_all.reshape(_K, _BLK, _DM), axis=0)
    VPE_s[...] = vv + pe

    inv = np.float32(1.0 / np.sqrt(_DM))
    s = jnp.zeros((_BLK, _DM), jnp.float32)
    for t in range(_K):
        s = s + jnp.exp((L_s[pl.ds(t * _BLK, _BLK), :] - m) * inv)
    rs = 1.0 / s
    racc = jnp.zeros((_BLK, _DM), jnp.float32)
    for t in range(_K):
        a_t = jnp.exp((L_s[pl.ds(t * _BLK, _BLK), :] - m) * inv) * rs
        attn_ref[0, :, t, :] = a_t
        racc = racc + a_t * VPE_s[pl.ds(t * _BLK, _BLK), :]
    res_ref[0] = _mm(racc, w_fc2_ref[...], _DF) + b_fc2_ref[...] + feat_blk_ref[0]


def kernel(xyz, features, w_fc1, b_fc1, w_fc2, b_fc2, w_d1, b_d1, w_d2, b_d2,
           w_g1, b_g1, w_g2, b_g2, w_q, w_k, w_v):
    xyzt = jnp.swapaxes(xyz, 1, 2)
    r2 = lambda v: v.reshape(1, -1)

    def wspec(shape):
        return pl.BlockSpec(shape, lambda b, j: (0,) * len(shape))

    in_specs = [
        pl.BlockSpec((1, _BLK, 3), lambda b, j: (b, j, 0)),
        pl.BlockSpec((1, _N, 3), lambda b, j: (b, 0, 0)),
        pl.BlockSpec((1, 3, _N), lambda b, j: (b, 0, 0)),
        pl.BlockSpec((1, _N, _DP), lambda b, j: (b, 0, 0)),
        pl.BlockSpec((1, _BLK, _DP), lambda b, j: (b, j, 0)),
        wspec((_DP, _DM)), wspec((1, _DM)),
        wspec((_DM, _DP)), wspec((1, _DP)),
        wspec((3, _DM)), wspec((1, _DM)),
        wspec((_DM, _DM)), wspec((1, _DM)),
        wspec((_DM, _DM)), wspec((1, _DM)),
        wspec((_DM, _DM)), wspec((1, _DM)),
        wspec((_DM, _DM)),
        wspec((_DM, _DM)),
        wspec((_DM, _DM)),
    ]
    out_specs = [
        pl.BlockSpec((1, _BLK, _DP), lambda b, j: (b, j, 0)),
        pl.BlockSpec((1, _BLK, _K, _DM), lambda b, j: (b, j, 0, 0)),
    ]
    out_shape = [
        jax.ShapeDtypeStruct((_B, _N, _DP), jnp.float32),
        jax.ShapeDtypeStruct((_B, _N, _K, _DM), jnp.float32),
    ]
    scratch_shapes = [
        pltpu.VMEM((_N, _GW), jnp.bfloat16),
        pltpu.VMEM((_N, _DM), jnp.float32),
        pltpu.VMEM((_DM, _DM), jnp.float32),
        pltpu.VMEM((1, _DM), jnp.float32),
        pltpu.VMEM((_BLK, _N), jnp.float32),
        pltpu.VMEM((_R, _N), jnp.bfloat16),
        pltpu.VMEM((_R, _DM), jnp.float32),
        pltpu.VMEM((_R, _DM), jnp.float32),
    ]
    res, attn = pl.pallas_call(
        _body,
        grid=(_B, _NBLK),
        in_specs=in_specs,
        out_specs=out_specs,
        out_shape=out_shape,
        scratch_shapes=scratch_shapes,
        compiler_params=pltpu.CompilerParams(
            dimension_semantics=("arbitrary", "arbitrary"),
            vmem_limit_bytes=64 * 1024 * 1024),
    )(xyz, xyz, xyzt, features,
      features, w_fc1, r2(b_fc1), w_fc2, r2(b_fc2), w_d1, r2(b_d1),
      w_d2, r2(b_d2), w_g1, r2(b_g1), w_g2, r2(b_g2), w_q, w_k, w_v)
    return res, attn
